# baseline (speedup 1.0000x reference)
import functools

import jax
import jax.numpy as jnp
from jax import lax
from jax.experimental import pallas as pl
from jax.experimental.pallas import tpu as pltpu

_B, _H, _W, _C = 16, 128, 128, 64
_CLASSES, _TOPK, _SCALE = 80, 40, 4
_HW = _H * _W
_LANES = 128
_BIG = 1 << 30


def _shift_y(a, d):
    z = jnp.zeros((1,) + a.shape[1:], a.dtype)
    if d == -1:
        return jnp.concatenate([z, a[:-1]], axis=0)
    if d == 1:
        return jnp.concatenate([a[1:], z], axis=0)
    return a


def _shift_x(a, d):
    z = jnp.zeros((a.shape[0], 1) + a.shape[2:], a.dtype)
    if d == -1:
        return jnp.concatenate([z, a[:, :-1]], axis=1)
    if d == 1:
        return jnp.concatenate([a[:, 1:], z], axis=1)
    return a


def _centernet_kernel(x_ref, w1c_ref, b1c_ref, w2hm_ref, b2hm_ref,
                      w2wr_ref, b2wr_ref, out_ref, m3_ref, mcs_ref):
    f32 = jnp.float32
    b = pl.program_id(0)
    slot = lax.rem(b, 2)
    prev = lax.rem(b + 1, 2)

    @pl.when(b < _B)
    def conv_phase():
        _conv_nms_pack(x_ref, w1c_ref, b1c_ref, w2hm_ref, b2hm_ref,
                       w2wr_ref, b2wr_ref, m3_ref, mcs_ref, slot)

    @pl.when(b > 0)
    def ext_phase():
        _extract_topk(out_ref, m3_ref, mcs_ref, prev)


def _rows(a, start, n):
    lo, hi = max(start, 0), min(start + n, _H)
    parts = []
    if lo > start:
        parts.append(jnp.zeros((lo - start,) + a.shape[1:], a.dtype))
    parts.append(a[lo:hi])
    if start + n > hi:
        parts.append(jnp.zeros((start + n - hi,) + a.shape[1:], a.dtype))
    return jnp.concatenate(parts, axis=0) if len(parts) > 1 else parts[0]


def _conv_nms_pack(x_ref, w1c_ref, b1c_ref, w2hm_ref, b2hm_ref,
                   w2wr_ref, b2wr_ref, m3_ref, mcs_ref, slot):
    f32 = jnp.float32
    x3 = x_ref[...]
    nq = 4
    qh = _H // nq
    for q in range(nq):
        r0 = q * qh
        hs, he = max(r0 - 1, 0), min(r0 + qh + 1, _H)
        n_h, o = he - hs, r0 - hs

        sq = jnp.concatenate(
            [_rows(x3, hs - 1, n_h), _rows(x3, hs, n_h),
             _rows(x3, hs + 1, n_h)], axis=2)
        s2 = sq.reshape(n_h * _W, 3 * _C)

        acc = None
        for kx in range(3):
            a = jnp.dot(s2, w1c_ref[kx], preferred_element_type=f32)
            a3 = _shift_x(a.reshape(n_h, _W, 3 * _C), kx - 1)
            acc = a3 if acc is None else acc + a3
        hq = jax.nn.relu(acc + b1c_ref[...])
        h2 = hq.reshape(n_h * _W, 3 * _C)

        heat = jax.nn.sigmoid(
            jnp.dot(h2[:, :_C], w2hm_ref[...], preferred_element_type=f32)
            + b2hm_ref[...])
        wr = (jnp.dot(h2[:, _C:], w2wr_ref[...], preferred_element_type=f32)
              + b2wr_ref[...])
        heat3 = heat.reshape(n_h, _W, _CLASSES)
        wr3 = wr.reshape(n_h, _W, 4)

        rowm = jnp.maximum(jnp.maximum(_shift_x(heat3, -1), heat3),
                           _shift_x(heat3, 1))
        pooled = jnp.maximum(jnp.maximum(_shift_y(rowm, -1), rowm),
                             _shift_y(rowm, 1))
        hown = heat3[o:o + qh]
        peaks = hown * (pooled[o:o + qh] == hown).astype(f32)

        pad = jnp.zeros((qh, _W, _LANES - _CLASSES - 4), f32)
        m3_ref[pl.ds(slot * _H + r0, qh)] = jnp.concatenate(
            [peaks, wr3[o:o + qh], pad], axis=2)
        mcs_ref[pl.ds(slot * _H + r0, qh)] = jnp.concatenate(
            [jnp.max(peaks, axis=1),
             jnp.full((qh, _LANES - _CLASSES), -1.0, f32)], axis=1)


def _extract_topk(out_ref, m3_ref, mcs_ref, prev):
    f32 = jnp.float32
    base = prev * _H
    mc0 = mcs_ref[pl.ds(base, _H)]

    yi = lax.broadcasted_iota(jnp.int32, (_H, _LANES), 0)
    ci = lax.broadcasted_iota(jnp.int32, (_H, _LANES), 1)
    xi_s = lax.broadcasted_iota(jnp.int32, (1, _W, _LANES), 1)
    ci_s = lax.broadcasted_iota(jnp.int32, (1, _W, _LANES), 2)
    li = lax.broadcasted_iota(jnp.int32, (1, _LANES), 1)
    li8 = lax.broadcasted_iota(jnp.int32, (1, 1, 8), 2)

    def body(k, mct):
        m = jnp.max(mct)
        p = jnp.min(jnp.where(mct == m, ci * _HW + yi * _W, _BIG))
        c_sel = p // _HW
        y_sel = (p - c_sel * _HW) // _W

        slab = m3_ref[pl.ds(base + y_sel, 1)]
        hit = (slab == m) & (ci_s == c_sel)
        x_sel = jnp.min(jnp.where(hit, xi_s, _BIG))

        at_x = xi_s == x_sel
        slab2 = jnp.where(at_x & (ci_s == c_sel), 0.0, slab)
        m3_ref[pl.ds(base + y_sel, 1)] = slab2
        mrow = jnp.max(jnp.where(ci_s < _CLASSES, slab2, -1.0),
                       axis=1)
        colv = jnp.sum(jnp.where(at_x, slab, 0.0), axis=1)
        mct = jnp.where(yi == y_sel, mrow, mct)

        wv = jnp.sum(jnp.where(li == _CLASSES + 0, colv, 0.0))
        hv = jnp.sum(jnp.where(li == _CLASSES + 1, colv, 0.0))
        ox = jnp.sum(jnp.where(li == _CLASSES + 2, colv, 0.0))
        oy = jnp.sum(jnp.where(li == _CLASSES + 3, colv, 0.0))

        cx = x_sel.astype(f32) + ox
        cy = y_sel.astype(f32) + oy
        sc = jnp.float32(_SCALE)
        vals = ((cx - 0.5 * wv) * sc, (cy - 0.5 * hv) * sc,
                (cx + 0.5 * wv) * sc, (cy + 0.5 * hv) * sc,
                c_sel.astype(f32), m)
        row = jnp.zeros((1, 1, 8), f32)
        for j, v in enumerate(vals):
            row = row + jnp.where(li8 == j, v, 0.0)
        out_ref[pl.ds(_TOPK - 1 - k, 1)] = row
        return mct

    lax.fori_loop(0, _TOPK, body, mc0)


@jax.jit
def kernel(x, hm_w1, hm_b1, hm_w2, hm_b2, wh_w1, wh_b1, wh_w2, wh_b2,
           reg_w1, reg_b1, reg_w2, reg_b2):
    f32 = jnp.float32
    w1c = jnp.concatenate(
        [jnp.transpose(w, (1, 0, 2, 3)).reshape(3, 3 * _C, _C)
         for w in (hm_w1, wh_w1, reg_w1)], axis=2)
    b1c = jnp.concatenate([hm_b1, wh_b1, reg_b1]).reshape(1, 1, 3 * _C)
    w2hm = hm_w2.reshape(_C, _CLASSES)
    b2hm = hm_b2.reshape(1, _CLASSES)
    z = jnp.zeros((_C, 2), f32)
    w2wr = jnp.concatenate(
        [jnp.concatenate([wh_w2.reshape(_C, 2), z], axis=1),
         jnp.concatenate([z, reg_w2.reshape(_C, 2)], axis=1)], axis=0)
    b2wr = jnp.concatenate([wh_b2, reg_b2]).reshape(1, 4)

    full = lambda *nd: pl.BlockSpec(nd, lambda b: (0,) * len(nd))
    out = pl.pallas_call(
        _centernet_kernel,
        grid=(_B + 1,),
        in_specs=[
            pl.BlockSpec((None, _H, _W, _C),
                         lambda b: (jnp.minimum(b, _B - 1), 0, 0, 0)),
            full(3, 3 * _C, 3 * _C),
            full(1, 1, 3 * _C),
            full(_C, _CLASSES),
            full(1, _CLASSES),
            full(2 * _C, 4),
            full(1, 4),
        ],
        out_specs=pl.BlockSpec((None, _TOPK, 1, 8),
                               lambda b: (jnp.maximum(b - 1, 0), 0, 0, 0)),
        out_shape=jax.ShapeDtypeStruct((_B, _TOPK, 1, 8), f32),
        scratch_shapes=[
            pltpu.VMEM((2 * _H, _W, _LANES), f32),
            pltpu.VMEM((2 * _H, _LANES), f32),
        ],
        compiler_params=pltpu.CompilerParams(
            dimension_semantics=("arbitrary",),
            vmem_limit_bytes=56 * 1024 * 1024,
        ),
    )(x, w1c, b1c, w2hm, b2hm, w2wr, b2wr)
    return out[:, :, 0, :6]


# speedup vs baseline: 1.0240x; 1.0240x over previous
import functools

import jax
import jax.numpy as jnp
from jax import lax
from jax.experimental import pallas as pl
from jax.experimental.pallas import tpu as pltpu

_B, _H, _W, _C = 16, 128, 128, 64
_CLASSES, _TOPK, _SCALE = 80, 40, 4
_HW = _H * _W
_LANES = 128
_BIG = 1 << 30


def _shift_y(a, d):
    z = jnp.zeros((1,) + a.shape[1:], a.dtype)
    if d == -1:
        return jnp.concatenate([z, a[:-1]], axis=0)
    if d == 1:
        return jnp.concatenate([a[1:], z], axis=0)
    return a


def _shift_x(a, d):
    z = jnp.zeros((a.shape[0], 1) + a.shape[2:], a.dtype)
    if d == -1:
        return jnp.concatenate([z, a[:, :-1]], axis=1)
    if d == 1:
        return jnp.concatenate([a[:, 1:], z], axis=1)
    return a


def _centernet_kernel(x_ref, w1c_ref, b1c_ref, w2hm_ref, b2hm_ref,
                      w2wr_ref, b2wr_ref, out_ref, m3_ref, mcs_ref):
    f32 = jnp.float32
    b = pl.program_id(0)
    slot = lax.rem(b, 2)
    prev = lax.rem(b + 1, 2)

    _conv_nms_pack(x_ref, w1c_ref, b1c_ref, w2hm_ref, b2hm_ref,
                   w2wr_ref, b2wr_ref, m3_ref, mcs_ref, slot)
    _extract_topk(out_ref, m3_ref, mcs_ref, prev)


def _rows(a, start, n):
    lo, hi = max(start, 0), min(start + n, _H)
    parts = []
    if lo > start:
        parts.append(jnp.zeros((lo - start,) + a.shape[1:], a.dtype))
    parts.append(a[lo:hi])
    if start + n > hi:
        parts.append(jnp.zeros((start + n - hi,) + a.shape[1:], a.dtype))
    return jnp.concatenate(parts, axis=0) if len(parts) > 1 else parts[0]


def _conv_nms_pack(x_ref, w1c_ref, b1c_ref, w2hm_ref, b2hm_ref,
                   w2wr_ref, b2wr_ref, m3_ref, mcs_ref, slot):
    f32 = jnp.float32
    x3 = x_ref[...]
    nq = 4
    qh = _H // nq
    for q in range(nq):
        r0 = q * qh
        hs, he = max(r0 - 1, 0), min(r0 + qh + 1, _H)
        n_h, o = he - hs, r0 - hs

        sq = jnp.concatenate(
            [_rows(x3, hs - 1, n_h), _rows(x3, hs, n_h),
             _rows(x3, hs + 1, n_h)], axis=2)
        s2 = sq.reshape(n_h * _W, 3 * _C)

        acc = None
        for kx in range(3):
            a = jnp.dot(s2, w1c_ref[kx], preferred_element_type=f32)
            a3 = _shift_x(a.reshape(n_h, _W, 3 * _C), kx - 1)
            acc = a3 if acc is None else acc + a3
        hq = jax.nn.relu(acc + b1c_ref[...])
        h2 = hq.reshape(n_h * _W, 3 * _C)

        heat = jax.nn.sigmoid(
            jnp.dot(h2[:, :_C], w2hm_ref[...], preferred_element_type=f32)
            + b2hm_ref[...])
        wr = (jnp.dot(h2[:, _C:], w2wr_ref[...], preferred_element_type=f32)
              + b2wr_ref[...])
        heat3 = heat.reshape(n_h, _W, _CLASSES)
        wr3 = wr.reshape(n_h, _W, 4)

        rowm = jnp.maximum(jnp.maximum(_shift_x(heat3, -1), heat3),
                           _shift_x(heat3, 1))
        pooled = jnp.maximum(jnp.maximum(_shift_y(rowm, -1), rowm),
                             _shift_y(rowm, 1))
        hown = heat3[o:o + qh]
        peaks = hown * (pooled[o:o + qh] == hown).astype(f32)

        pad = jnp.zeros((qh, _W, _LANES - _CLASSES - 4), f32)
        m3_ref[pl.ds(slot * _H + r0, qh)] = jnp.concatenate(
            [peaks, wr3[o:o + qh], pad], axis=2)
        mcs_ref[pl.ds(slot * _H + r0, qh)] = jnp.concatenate(
            [jnp.max(peaks, axis=1),
             jnp.full((qh, _LANES - _CLASSES), -1.0, f32)], axis=1)


def _extract_topk(out_ref, m3_ref, mcs_ref, prev):
    f32 = jnp.float32
    base = prev * _H
    mc0 = mcs_ref[pl.ds(base, _H)]

    yi = lax.broadcasted_iota(jnp.int32, (_H, _LANES), 0)
    ci = lax.broadcasted_iota(jnp.int32, (_H, _LANES), 1)
    xi_s = lax.broadcasted_iota(jnp.int32, (1, _W, _LANES), 1)
    ci_s = lax.broadcasted_iota(jnp.int32, (1, _W, _LANES), 2)
    li = lax.broadcasted_iota(jnp.int32, (1, _LANES), 1)
    li8 = lax.broadcasted_iota(jnp.int32, (1, 1, 8), 2)

    def body(k, mct):
        m = jnp.max(mct)
        p = jnp.min(jnp.where(mct == m, ci * _HW + yi * _W, _BIG))
        c_sel = jnp.minimum(p // _HW, _LANES - 1)
        y_sel = jnp.minimum((p - (p // _HW) * _HW) // _W, _H - 1)

        slab = m3_ref[pl.ds(base + y_sel, 1)]
        hit = (slab == m) & (ci_s == c_sel)
        x_sel = jnp.min(jnp.where(hit, xi_s, _BIG))

        at_x = xi_s == x_sel
        slab2 = jnp.where(at_x & (ci_s == c_sel), 0.0, slab)
        m3_ref[pl.ds(base + y_sel, 1)] = slab2
        mrow = jnp.max(jnp.where(ci_s < _CLASSES, slab2, -1.0),
                       axis=1)
        colv = jnp.sum(jnp.where(at_x, slab, 0.0), axis=1)
        mct = jnp.where(yi == y_sel, mrow, mct)

        wv = jnp.sum(jnp.where(li == _CLASSES + 0, colv, 0.0))
        hv = jnp.sum(jnp.where(li == _CLASSES + 1, colv, 0.0))
        ox = jnp.sum(jnp.where(li == _CLASSES + 2, colv, 0.0))
        oy = jnp.sum(jnp.where(li == _CLASSES + 3, colv, 0.0))

        cx = x_sel.astype(f32) + ox
        cy = y_sel.astype(f32) + oy
        sc = jnp.float32(_SCALE)
        vals = ((cx - 0.5 * wv) * sc, (cy - 0.5 * hv) * sc,
                (cx + 0.5 * wv) * sc, (cy + 0.5 * hv) * sc,
                c_sel.astype(f32), m)
        row = jnp.zeros((1, 1, 8), f32)
        for j, v in enumerate(vals):
            row = row + jnp.where(li8 == j, v, 0.0)
        out_ref[pl.ds(_TOPK - 1 - k, 1)] = row
        return mct

    mct = mc0
    for k in range(_TOPK):
        mct = body(k, mct)


@jax.jit
def kernel(x, hm_w1, hm_b1, hm_w2, hm_b2, wh_w1, wh_b1, wh_w2, wh_b2,
           reg_w1, reg_b1, reg_w2, reg_b2):
    f32 = jnp.float32
    w1c = jnp.concatenate(
        [jnp.transpose(w, (1, 0, 2, 3)).reshape(3, 3 * _C, _C)
         for w in (hm_w1, wh_w1, reg_w1)], axis=2)
    b1c = jnp.concatenate([hm_b1, wh_b1, reg_b1]).reshape(1, 1, 3 * _C)
    w2hm = hm_w2.reshape(_C, _CLASSES)
    b2hm = hm_b2.reshape(1, _CLASSES)
    z = jnp.zeros((_C, 2), f32)
    w2wr = jnp.concatenate(
        [jnp.concatenate([wh_w2.reshape(_C, 2), z], axis=1),
         jnp.concatenate([z, reg_w2.reshape(_C, 2)], axis=1)], axis=0)
    b2wr = jnp.concatenate([wh_b2, reg_b2]).reshape(1, 4)

    full = lambda *nd: pl.BlockSpec(nd, lambda b: (0,) * len(nd))
    out = pl.pallas_call(
        _centernet_kernel,
        grid=(_B + 1,),
        in_specs=[
            pl.BlockSpec((None, _H, _W, _C),
                         lambda b: (jnp.minimum(b, _B - 1), 0, 0, 0)),
            full(3, 3 * _C, 3 * _C),
            full(1, 1, 3 * _C),
            full(_C, _CLASSES),
            full(1, _CLASSES),
            full(2 * _C, 4),
            full(1, 4),
        ],
        out_specs=pl.BlockSpec((None, _TOPK, 1, 8),
                               lambda b: (jnp.maximum(b - 1, 0), 0, 0, 0)),
        out_shape=jax.ShapeDtypeStruct((_B, _TOPK, 1, 8), f32),
        scratch_shapes=[
            pltpu.VMEM((2 * _H, _W, _LANES), f32),
            pltpu.VMEM((2 * _H, _LANES), f32),
        ],
        compiler_params=pltpu.CompilerParams(
            dimension_semantics=("arbitrary",),
            vmem_limit_bytes=56 * 1024 * 1024,
        ),
    )(x, w1c, b1c, w2hm, b2hm, w2wr, b2wr)
    return out[:, :, 0, :6]


# speedup vs baseline: 1.0306x; 1.0064x over previous
import functools

import jax
import jax.numpy as jnp
from jax import lax
from jax.experimental import pallas as pl
from jax.experimental.pallas import tpu as pltpu

_B, _H, _W, _C = 16, 128, 128, 64
_CLASSES, _TOPK, _SCALE = 80, 40, 4
_HW = _H * _W
_LANES = 128
_BIG = 1 << 30


def _shift_y(a, d):
    z = jnp.zeros((1,) + a.shape[1:], a.dtype)
    if d == -1:
        return jnp.concatenate([z, a[:-1]], axis=0)
    if d == 1:
        return jnp.concatenate([a[1:], z], axis=0)
    return a


def _shift_x(a, d):
    z = jnp.zeros((a.shape[0], 1) + a.shape[2:], a.dtype)
    if d == -1:
        return jnp.concatenate([z, a[:, :-1]], axis=1)
    if d == 1:
        return jnp.concatenate([a[:, 1:], z], axis=1)
    return a


def _centernet_kernel(x_ref, w1c_ref, b1c_ref, w2hm_ref, b2hm_ref,
                      w2wr_ref, b2wr_ref, out_ref, m3_ref, mcs_ref):
    f32 = jnp.float32
    b = pl.program_id(0)
    slot = lax.rem(b, 2)
    prev = lax.rem(b + 1, 2)

    _conv_nms_pack(x_ref, w1c_ref, b1c_ref, w2hm_ref, b2hm_ref,
                   w2wr_ref, b2wr_ref, m3_ref, mcs_ref, slot)
    _extract_topk(out_ref, m3_ref, mcs_ref, prev)


def _rows(a, start, n):
    lo, hi = max(start, 0), min(start + n, _H)
    parts = []
    if lo > start:
        parts.append(jnp.zeros((lo - start,) + a.shape[1:], a.dtype))
    parts.append(a[lo:hi])
    if start + n > hi:
        parts.append(jnp.zeros((start + n - hi,) + a.shape[1:], a.dtype))
    return jnp.concatenate(parts, axis=0) if len(parts) > 1 else parts[0]


def _conv_nms_pack(x_ref, w1c_ref, b1c_ref, w2hm_ref, b2hm_ref,
                   w2wr_ref, b2wr_ref, m3_ref, mcs_ref, slot):
    f32 = jnp.float32
    x3 = x_ref[...]
    nq = 2
    qh = _H // nq
    for q in range(nq):
        r0 = q * qh
        hs, he = max(r0 - 1, 0), min(r0 + qh + 1, _H)
        n_h, o = he - hs, r0 - hs

        sq = jnp.concatenate(
            [_rows(x3, hs - 1, n_h), _rows(x3, hs, n_h),
             _rows(x3, hs + 1, n_h)], axis=2)
        s2 = sq.reshape(n_h * _W, 3 * _C)

        acc = None
        for kx in range(3):
            a = jnp.dot(s2, w1c_ref[kx], preferred_element_type=f32)
            a3 = _shift_x(a.reshape(n_h, _W, 3 * _C), kx - 1)
            acc = a3 if acc is None else acc + a3
        hq = jax.nn.relu(acc + b1c_ref[...])
        h2 = hq.reshape(n_h * _W, 3 * _C)

        heat = jax.nn.sigmoid(
            jnp.dot(h2[:, :_C], w2hm_ref[...], preferred_element_type=f32)
            + b2hm_ref[...])
        wr = (jnp.dot(h2[:, _C:], w2wr_ref[...], preferred_element_type=f32)
              + b2wr_ref[...])
        heat3 = heat.reshape(n_h, _W, _CLASSES)
        wr3 = wr.reshape(n_h, _W, 4)

        rowm = jnp.maximum(jnp.maximum(_shift_x(heat3, -1), heat3),
                           _shift_x(heat3, 1))
        pooled = jnp.maximum(jnp.maximum(_shift_y(rowm, -1), rowm),
                             _shift_y(rowm, 1))
        hown = heat3[o:o + qh]
        peaks = hown * (pooled[o:o + qh] == hown).astype(f32)

        pad = jnp.zeros((qh, _W, _LANES - _CLASSES - 4), f32)
        m3_ref[pl.ds(slot * _H + r0, qh)] = jnp.concatenate(
            [peaks, wr3[o:o + qh], pad], axis=2)
        mcs_ref[pl.ds(slot * _H + r0, qh)] = jnp.concatenate(
            [jnp.max(peaks, axis=1),
             jnp.full((qh, _LANES - _CLASSES), -1.0, f32)], axis=1)


def _extract_topk(out_ref, m3_ref, mcs_ref, prev):
    f32 = jnp.float32
    base = prev * _H
    mc0 = mcs_ref[pl.ds(base, _H)]

    yi = lax.broadcasted_iota(jnp.int32, (_H, _LANES), 0)
    ci = lax.broadcasted_iota(jnp.int32, (_H, _LANES), 1)
    xi_s = lax.broadcasted_iota(jnp.int32, (1, _W, _LANES), 1)
    ci_s = lax.broadcasted_iota(jnp.int32, (1, _W, _LANES), 2)
    li = lax.broadcasted_iota(jnp.int32, (1, _LANES), 1)
    li8 = lax.broadcasted_iota(jnp.int32, (1, 1, 8), 2)

    def body(k, mct):
        m = jnp.max(mct)
        p = jnp.min(jnp.where(mct == m, ci * _HW + yi * _W, _BIG))
        c_sel = jnp.minimum(p // _HW, _LANES - 1)
        y_sel = jnp.minimum((p - (p // _HW) * _HW) // _W, _H - 1)

        slab = m3_ref[pl.ds(base + y_sel, 1)]
        hit = (slab == m) & (ci_s == c_sel)
        x_sel = jnp.min(jnp.where(hit, xi_s, _BIG))

        at_x = xi_s == x_sel
        slab2 = jnp.where(at_x & (ci_s == c_sel), 0.0, slab)
        m3_ref[pl.ds(base + y_sel, 1)] = slab2
        mrow = jnp.max(jnp.where(ci_s < _CLASSES, slab2, -1.0),
                       axis=1)
        colv = jnp.sum(jnp.where(at_x, slab, 0.0), axis=1)
        mct = jnp.where(yi == y_sel, mrow, mct)

        wv = jnp.sum(jnp.where(li == _CLASSES + 0, colv, 0.0))
        hv = jnp.sum(jnp.where(li == _CLASSES + 1, colv, 0.0))
        ox = jnp.sum(jnp.where(li == _CLASSES + 2, colv, 0.0))
        oy = jnp.sum(jnp.where(li == _CLASSES + 3, colv, 0.0))

        cx = x_sel.astype(f32) + ox
        cy = y_sel.astype(f32) + oy
        sc = jnp.float32(_SCALE)
        vals = ((cx - 0.5 * wv) * sc, (cy - 0.5 * hv) * sc,
                (cx + 0.5 * wv) * sc, (cy + 0.5 * hv) * sc,
                c_sel.astype(f32), m)
        row = jnp.zeros((1, 1, 8), f32)
        for j, v in enumerate(vals):
            row = row + jnp.where(li8 == j, v, 0.0)
        out_ref[pl.ds(_TOPK - 1 - k, 1)] = row
        return mct

    mct = mc0
    for k in range(_TOPK):
        mct = body(k, mct)


@jax.jit
def kernel(x, hm_w1, hm_b1, hm_w2, hm_b2, wh_w1, wh_b1, wh_w2, wh_b2,
           reg_w1, reg_b1, reg_w2, reg_b2):
    f32 = jnp.float32
    w1c = jnp.concatenate(
        [jnp.transpose(w, (1, 0, 2, 3)).reshape(3, 3 * _C, _C)
         for w in (hm_w1, wh_w1, reg_w1)], axis=2)
    b1c = jnp.concatenate([hm_b1, wh_b1, reg_b1]).reshape(1, 1, 3 * _C)
    w2hm = hm_w2.reshape(_C, _CLASSES)
    b2hm = hm_b2.reshape(1, _CLASSES)
    z = jnp.zeros((_C, 2), f32)
    w2wr = jnp.concatenate(
        [jnp.concatenate([wh_w2.reshape(_C, 2), z], axis=1),
         jnp.concatenate([z, reg_w2.reshape(_C, 2)], axis=1)], axis=0)
    b2wr = jnp.concatenate([wh_b2, reg_b2]).reshape(1, 4)

    full = lambda *nd: pl.BlockSpec(nd, lambda b: (0,) * len(nd))
    out = pl.pallas_call(
        _centernet_kernel,
        grid=(_B + 1,),
        in_specs=[
            pl.BlockSpec((None, _H, _W, _C),
                         lambda b: (jnp.minimum(b, _B - 1), 0, 0, 0)),
            full(3, 3 * _C, 3 * _C),
            full(1, 1, 3 * _C),
            full(_C, _CLASSES),
            full(1, _CLASSES),
            full(2 * _C, 4),
            full(1, 4),
        ],
        out_specs=pl.BlockSpec((None, _TOPK, 1, 8),
                               lambda b: (jnp.maximum(b - 1, 0), 0, 0, 0)),
        out_shape=jax.ShapeDtypeStruct((_B, _TOPK, 1, 8), f32),
        scratch_shapes=[
            pltpu.VMEM((2 * _H, _W, _LANES), f32),
            pltpu.VMEM((2 * _H, _LANES), f32),
        ],
        compiler_params=pltpu.CompilerParams(
            dimension_semantics=("arbitrary",),
            vmem_limit_bytes=56 * 1024 * 1024,
        ),
    )(x, w1c, b1c, w2hm, b2hm, w2wr, b2wr)
    return out[:, :, 0, :6]
